# baseline (speedup 1.0000x reference)
import functools

import jax
import jax.numpy as jnp
from jax import lax
from jax.experimental import pallas as pl
from jax.experimental.pallas import tpu as pltpu
from jax.experimental.pallas import tpu_sc as plsc

_GRP = 40
_CB = 2
_NBUF = 2


@functools.lru_cache(maxsize=None)
def _build(B, H, V, F, num_cores, num_subcores):
    NW = num_cores * num_subcores
    BPW = B // NW
    K = (_CB * H) // _GRP
    NCHUNK = BPW // _CB
    assert NCHUNK >= 2 and NCHUNK % 2 == 0 and H % _GRP == 0

    mesh = plsc.VectorSubcoreMesh(core_axis_name="c", subcore_axis_name="s")

    @functools.partial(
        pl.kernel,
        mesh=mesh,
        compiler_params=pltpu.CompilerParams(use_tc_tiling_on_sc=False),
        out_type=jax.ShapeDtypeStruct((B, H, F), jnp.float32),
        scratch_types=[
            pltpu.VMEM((BPW, H), jnp.int32),
            pltpu.VMEM((_NBUF * _CB, H, F), jnp.float32),
            pltpu.SemaphoreType.DMA,
            pltpu.SemaphoreType.DMA,
            pltpu.SemaphoreType.DMA,
            pltpu.SemaphoreType.DMA,
        ],
    )
    def body(idx_hbm, table_hbm, out_hbm, idx_v, rows_v,
             sem_g0, sem_g1, sem_o0, sem_o1):
        wid = lax.axis_index("s") * num_cores + lax.axis_index("c")
        sem_g = (sem_g0, sem_g1)
        sem_o = (sem_o0, sem_o1)
        pltpu.sync_copy(idx_hbm.at[pl.ds(wid * BPW, BPW)], idx_v)

        def gathers(g, b):
            descs = []
            for j in range(K):
                flat = j * _GRP
                bi = flat // H
                hi = flat % H
                descs.append(pltpu.make_async_copy(
                    table_hbm.at[idx_v.at[g * _CB + bi, pl.ds(hi, _GRP)]],
                    rows_v.at[b * _CB + bi, pl.ds(hi, _GRP)],
                    sem_g[b],
                ))
            return descs

        def out_copy(g, b):
            return pltpu.make_async_copy(
                rows_v.at[pl.ds(b * _CB, _CB)],
                out_hbm.at[pl.ds(wid * BPW + g * _CB, _CB)],
                sem_o[b],
            )

        for d in gathers(0, 0):
            d.start()
        for d in gathers(1, 1):
            d.start()
        for d in gathers(0, 0):
            d.wait()
        out_copy(0, 0).start()

        def main(go, carry):
            for off in range(2):
                g = 2 * go + 1 + off
                b = 1 - off
                out_copy(g - 1, 1 - b).wait()
                for d in gathers(g + 1, 1 - b):
                    d.start()
                for d in gathers(g, b):
                    d.wait()
                out_copy(g, b).start()
            return carry

        lax.fori_loop(0, (NCHUNK - 2) // 2, main, 0)

        out_copy(NCHUNK - 2, 0).wait()
        for d in gathers(NCHUNK - 1, 1):
            d.wait()
        out_copy(NCHUNK - 1, 1).start()
        out_copy(NCHUNK - 1, 1).wait()

    return body


def kernel(inputs, embedding):
    B, H = inputs.shape
    V, F = embedding.shape
    info = plsc.get_sparse_core_info()
    idx = inputs.astype(jnp.int32)
    return _build(B, H, V, F, info.num_cores, info.num_subcores)(idx, embedding)
